# baseline (speedup 1.0000x reference)
import jax
import jax.numpy as jnp
from jax import lax
from jax.experimental import pallas as pl
from jax.experimental.pallas import tpu as pltpu
from jax.experimental.pallas import tpu_sc as plsc

BATCH = 4096
SEQ = 500
EMB = 64
VOCAB = 1000

SEQ_PAD = 512
EROW = 128
NW = 32
B_PER_W = BATCH // NW
GROUP = 32
N_CHUNK = SEQ_PAD // 128


def _sc_body(data_hbm, e_hbm, x_hbm, espm, dbuf, xb0, xb1, yb0, yb1,
             semE, sg0, sg1, sw0, sw1):
    xbufs = (xb0, xb1)
    ybufs = (yb0, yb1)
    gsems = (sg0, sg1)
    wsems = (sw0, sw1)
    wid = lax.axis_index("s") * 2 + lax.axis_index("c")
    base = wid * B_PER_W

    @pl.when(lax.axis_index("s") == 0)
    def _():
        pltpu.async_copy(e_hbm, espm, semE).wait()

    plsc.subcore_barrier()

    def xdst(b, c):
        return x_hbm.at[base + b, pl.ds(c * 64, 64)]

    def compact(c):
        xb, yb = xbufs[c % 2], ybufs[c % 2]

        def tbody(t, carry):
            for q in range(4):
                yb[t, pl.ds(q * 16, 16)] = xb[2 * t, pl.ds(q * 16, 16)]
                yb[t, pl.ds(64 + q * 16, 16)] = \
                    xb[2 * t + 1, pl.ds(q * 16, 16)]
            return carry

        lax.fori_loop(0, 64, tbody, 0)

    def do_row(b, r, first):
        for c in range(N_CHUNK):
            if not (first and c < 2):
                pltpu.make_async_copy(ybufs[c % 2], xdst(b, c),
                                      wsems[c % 2]).wait()
            pltpu.async_copy(espm.at[dbuf.at[r, c]], xbufs[c % 2],
                             gsems[c % 2]).wait()
            compact(c)
            pltpu.async_copy(ybufs[c % 2], xdst(b, c), wsems[c % 2])

    for g in range(B_PER_W // GROUP):
        pltpu.sync_copy(data_hbm.at[pl.ds(base + g * GROUP, GROUP)], dbuf)
        if g == 0:
            do_row(0, 0, first=True)
            lax.fori_loop(
                1, GROUP, lambda r, _: (do_row(r, r, False), 0)[1], 0)
        else:
            lax.fori_loop(
                0, GROUP,
                lambda r, _, gg=g: (do_row(gg * GROUP + r, r, False), 0)[1],
                0)
    pltpu.make_async_copy(ybufs[0], xdst(0, 0), wsems[0]).wait()
    pltpu.make_async_copy(ybufs[1], xdst(0, 1), wsems[1]).wait()


def _sc_gather(data_p, E_pad):
    mesh = plsc.VectorSubcoreMesh(core_axis_name="c", subcore_axis_name="s")
    f = pl.kernel(
        _sc_body,
        mesh=mesh,
        out_type=jax.ShapeDtypeStruct((BATCH, SEQ_PAD * EMB // 128, 128),
                                      jnp.float32),
        scratch_types=[
            pltpu.VMEM_SHARED((VOCAB, EROW), jnp.float32),
            pltpu.VMEM((GROUP, N_CHUNK, 128), jnp.int32),
            pltpu.VMEM((128, EROW), jnp.float32),
            pltpu.VMEM((128, EROW), jnp.float32),
            pltpu.VMEM((64, 128), jnp.float32),
            pltpu.VMEM((64, 128), jnp.float32),
            pltpu.SemaphoreType.DMA,
            pltpu.SemaphoreType.DMA,
            pltpu.SemaphoreType.DMA,
            pltpu.SemaphoreType.DMA,
            pltpu.SemaphoreType.DMA,
        ],
    )
    return f(data_p, E_pad)


B_TILE = 512
M_CHUNK = 32
N_M = SEQ_PAD * EMB // 128


def _mlp_body(x_ref, w1_ref, b1_ref, w2_ref, b2_ref, w3_ref, b3_ref,
              mu_ref, sc_ref, acc_ref):
    k = pl.program_id(1)
    nk = pl.num_programs(1)

    @pl.when(k == 0)
    def _():
        acc_ref[...] = jnp.zeros_like(acc_ref)

    xb = x_ref[...].reshape(B_TILE, M_CHUNK * 128)
    wb = w1_ref[...].reshape(M_CHUNK * 128, 128)
    acc_ref[...] += jnp.dot(xb, wb, preferred_element_type=jnp.float32)

    @pl.when(k == nk - 1)
    def _():
        h1 = jnp.maximum(acc_ref[...] + b1_ref[...], 0.0)
        h2 = jnp.maximum(
            jnp.dot(h1, w2_ref[...], preferred_element_type=jnp.float32)
            + b2_ref[...], 0.0)
        out = jnp.dot(h2, w3_ref[...],
                      preferred_element_type=jnp.float32) + b3_ref[...]
        mu_ref[...] = out[:, :2]
        sc_ref[...] = jnp.logaddexp(out[:, 2:] - 5.0, 0.0)


def _mlp(x, W1p, b1, W2, b2, W3, b3):
    nb = BATCH // B_TILE
    nk = N_M // M_CHUNK
    mu, scale = pl.pallas_call(
        _mlp_body,
        grid=(nb, nk),
        in_specs=[
            pl.BlockSpec((B_TILE, M_CHUNK, 128), lambda b, k: (b, k, 0)),
            pl.BlockSpec((M_CHUNK, 128, 128), lambda b, k: (k, 0, 0)),
            pl.BlockSpec((128,), lambda b, k: (0,)),
            pl.BlockSpec((128, 64), lambda b, k: (0, 0)),
            pl.BlockSpec((64,), lambda b, k: (0,)),
            pl.BlockSpec((64, 4), lambda b, k: (0, 0)),
            pl.BlockSpec((4,), lambda b, k: (0,)),
        ],
        out_specs=[
            pl.BlockSpec((B_TILE, 2), lambda b, k: (b, 0)),
            pl.BlockSpec((B_TILE, 2), lambda b, k: (b, 0)),
        ],
        out_shape=[
            jax.ShapeDtypeStruct((BATCH, 2), jnp.float32),
            jax.ShapeDtypeStruct((BATCH, 2), jnp.float32),
        ],
        scratch_shapes=[pltpu.VMEM((B_TILE, 128), jnp.float32)],
    )(x, W1p, b1, W2, b2, W3, b3)
    return mu, scale


def kernel(data, E, W1, b1, W2, b2, W3, b3):
    data_p = jnp.pad(data, ((0, 0), (0, SEQ_PAD - SEQ)))
    data_p = data_p.reshape(BATCH, N_CHUNK, 128)
    E_pad = jnp.pad(E, ((0, 0), (0, EROW - EMB)))
    x = _sc_gather(data_p, E_pad)
    W1p = jnp.pad(W1, ((0, (SEQ_PAD - SEQ) * EMB), (0, 0)))
    W1p = W1p.reshape(N_M, 128, 128)
    return _mlp(x, W1p, b1, W2, b2, W3, b3)


# speedup vs baseline: 1.8074x; 1.8074x over previous
import jax
import jax.numpy as jnp
from jax import lax
from jax.experimental import pallas as pl
from jax.experimental.pallas import tpu as pltpu
from jax.experimental.pallas import tpu_sc as plsc

BATCH = 4096
SEQ = 500
EMB = 64
VOCAB = 1000

SEQ_PAD = 512
EROW = 128
NW = 32
HALF = BATCH // 2
B_PER_W = HALF // NW
GROUP = 32
N_CHUNK = SEQ_PAD // 128


def _sc_body(data_hbm, e_hbm, x_hbm, espm, dbuf, xb0, xb1,
             semE, sg0, sg1, sw0, sw1):
    xbufs = (xb0, xb1)
    gsems = (sg0, sg1)
    wsems = (sw0, sw1)
    wid = lax.axis_index("s") * 2 + lax.axis_index("c")
    base = wid * B_PER_W

    @pl.when(lax.axis_index("s") == 0)
    def _():
        pltpu.async_copy(e_hbm, espm, semE).wait()

    plsc.subcore_barrier()

    def xsrc(c):
        return xbufs[c % 2]

    def xdst(b, c):
        return x_hbm.at[base + b, pl.ds(c * 128, 128)]

    def do_row(b, r, first):
        for c in range(N_CHUNK):
            if not (first and c < 2):
                pltpu.make_async_copy(xsrc(c), xdst(b, c),
                                      wsems[c % 2]).wait()
            pltpu.async_copy(espm.at[dbuf.at[r, c]], xbufs[c % 2],
                             gsems[c % 2]).wait()
            pltpu.async_copy(xsrc(c), xdst(b, c), wsems[c % 2])

    for g in range(B_PER_W // GROUP):
        pltpu.sync_copy(data_hbm.at[pl.ds(base + g * GROUP, GROUP)], dbuf)
        if g == 0:
            do_row(0, 0, first=True)
            lax.fori_loop(
                1, GROUP, lambda r, _: (do_row(r, r, False), 0)[1], 0)
        else:
            lax.fori_loop(
                0, GROUP,
                lambda r, _, gg=g: (do_row(gg * GROUP + r, r, False), 0)[1],
                0)
    pltpu.make_async_copy(xsrc(0), xdst(0, 0), wsems[0]).wait()
    pltpu.make_async_copy(xsrc(1), xdst(0, 1), wsems[1]).wait()


def _sc_gather(data_p, E_pad):
    mesh = plsc.VectorSubcoreMesh(core_axis_name="c", subcore_axis_name="s")
    f = pl.kernel(
        _sc_body,
        mesh=mesh,
        out_type=jax.ShapeDtypeStruct((HALF, SEQ_PAD, EROW), jnp.float32),
        scratch_types=[
            pltpu.VMEM_SHARED((VOCAB, EROW), jnp.float32),
            pltpu.VMEM((GROUP, N_CHUNK, 128), jnp.int32),
            pltpu.VMEM((128, EROW), jnp.float32),
            pltpu.VMEM((128, EROW), jnp.float32),
            pltpu.SemaphoreType.DMA,
            pltpu.SemaphoreType.DMA,
            pltpu.SemaphoreType.DMA,
            pltpu.SemaphoreType.DMA,
            pltpu.SemaphoreType.DMA,
        ],
    )
    return f(data_p, E_pad)


B_TILE = 512
S_CHUNK = 32


def _mlp_body(x_ref, w1_ref, b1_ref, w2_ref, b2_ref, w3_ref, b3_ref,
              mu_ref, sc_ref, acc_ref):
    k = pl.program_id(1)
    nk = pl.num_programs(1)

    @pl.when(k == 0)
    def _():
        acc_ref[...] = jnp.zeros_like(acc_ref)

    xb = x_ref[...].reshape(B_TILE, S_CHUNK * EROW)
    wb = w1_ref[...].reshape(S_CHUNK * EROW, 128)
    acc_ref[...] += jnp.dot(xb, wb, preferred_element_type=jnp.float32)

    @pl.when(k == nk - 1)
    def _():
        h1 = jnp.maximum(acc_ref[...] + b1_ref[...], 0.0)
        h2 = jnp.maximum(
            jnp.dot(h1, w2_ref[...], preferred_element_type=jnp.float32)
            + b2_ref[...], 0.0)
        out = jnp.dot(h2, w3_ref[...],
                      preferred_element_type=jnp.float32) + b3_ref[...]
        mu_ref[...] = out[:, :2]
        sc_ref[...] = jnp.logaddexp(out[:, 2:] - 5.0, 0.0)


def _mlp(x, W1p, b1, W2, b2, W3, b3):
    nb = HALF // B_TILE
    nk = SEQ_PAD // S_CHUNK
    mu, scale = pl.pallas_call(
        _mlp_body,
        grid=(nb, nk),
        in_specs=[
            pl.BlockSpec((B_TILE, S_CHUNK, EROW), lambda b, k: (b, k, 0)),
            pl.BlockSpec((S_CHUNK, EROW, 128), lambda b, k: (k, 0, 0)),
            pl.BlockSpec((128,), lambda b, k: (0,)),
            pl.BlockSpec((128, 64), lambda b, k: (0, 0)),
            pl.BlockSpec((64,), lambda b, k: (0,)),
            pl.BlockSpec((64, 4), lambda b, k: (0, 0)),
            pl.BlockSpec((4,), lambda b, k: (0,)),
        ],
        out_specs=[
            pl.BlockSpec((B_TILE, 2), lambda b, k: (b, 0)),
            pl.BlockSpec((B_TILE, 2), lambda b, k: (b, 0)),
        ],
        out_shape=[
            jax.ShapeDtypeStruct((HALF, 2), jnp.float32),
            jax.ShapeDtypeStruct((HALF, 2), jnp.float32),
        ],
        scratch_shapes=[pltpu.VMEM((B_TILE, 128), jnp.float32)],
    )(x, W1p, b1, W2, b2, W3, b3)
    return mu, scale


def kernel(data, E, W1, b1, W2, b2, W3, b3):
    data_p = jnp.pad(data, ((0, 0), (0, SEQ_PAD - SEQ)))
    data_p = data_p.reshape(BATCH, N_CHUNK, 128)
    E_pad = jnp.pad(E, ((0, 0), (0, EROW - EMB)))
    W1pp = jnp.pad(W1.reshape(SEQ, EMB, 128),
                   ((0, SEQ_PAD - SEQ), (0, EROW - EMB), (0, 0)))
    x1 = _sc_gather(data_p[:HALF], E_pad)
    x2 = _sc_gather(data_p[HALF:], E_pad)
    mu1, sc1 = _mlp(x1, W1pp, b1, W2, b2, W3, b3)
    mu2, sc2 = _mlp(x2, W1pp, b1, W2, b2, W3, b3)
    return (jnp.concatenate([mu1, mu2]), jnp.concatenate([sc1, sc2]))


# speedup vs baseline: 1.8372x; 1.0165x over previous
import jax
import jax.numpy as jnp
from jax import lax
from jax.experimental import pallas as pl
from jax.experimental.pallas import tpu as pltpu
from jax.experimental.pallas import tpu_sc as plsc

BATCH = 4096
SEQ = 500
EMB = 64
VOCAB = 1000

SEQ_PAD = 512
EROW = 128
NW = 32
HALF = BATCH // 4
B_PER_W = HALF // NW
GROUP = 32
N_CHUNK = SEQ_PAD // 128


def _sc_body(data_hbm, e_hbm, x_hbm, espm, dbuf, xb0, xb1,
             semE, sg0, sg1, sw0, sw1):
    xbufs = (xb0, xb1)
    gsems = (sg0, sg1)
    wsems = (sw0, sw1)
    wid = lax.axis_index("s") * 2 + lax.axis_index("c")
    base = wid * B_PER_W

    @pl.when(lax.axis_index("s") == 0)
    def _():
        pltpu.async_copy(e_hbm, espm, semE).wait()

    plsc.subcore_barrier()

    def xsrc(c):
        return xbufs[c % 2]

    def xdst(b, c):
        return x_hbm.at[base + b, pl.ds(c * 128, 128)]

    def do_row(b, r, first):
        for c in range(N_CHUNK):
            if not (first and c < 2):
                pltpu.make_async_copy(xsrc(c), xdst(b, c),
                                      wsems[c % 2]).wait()
            pltpu.async_copy(espm.at[dbuf.at[r, c]], xbufs[c % 2],
                             gsems[c % 2]).wait()
            pltpu.async_copy(xsrc(c), xdst(b, c), wsems[c % 2])

    for g in range(B_PER_W // GROUP):
        pltpu.sync_copy(data_hbm.at[pl.ds(base + g * GROUP, GROUP)], dbuf)
        if g == 0:
            do_row(0, 0, first=True)
            lax.fori_loop(
                1, GROUP, lambda r, _: (do_row(r, r, False), 0)[1], 0)
        else:
            lax.fori_loop(
                0, GROUP,
                lambda r, _, gg=g: (do_row(gg * GROUP + r, r, False), 0)[1],
                0)
    pltpu.make_async_copy(xsrc(0), xdst(0, 0), wsems[0]).wait()
    pltpu.make_async_copy(xsrc(1), xdst(0, 1), wsems[1]).wait()


def _sc_gather(data_p, E_pad):
    mesh = plsc.VectorSubcoreMesh(core_axis_name="c", subcore_axis_name="s")
    f = pl.kernel(
        _sc_body,
        mesh=mesh,
        out_type=jax.ShapeDtypeStruct((HALF, SEQ_PAD, EROW), jnp.float32),
        scratch_types=[
            pltpu.VMEM_SHARED((VOCAB, EROW), jnp.float32),
            pltpu.VMEM((GROUP, N_CHUNK, 128), jnp.int32),
            pltpu.VMEM((128, EROW), jnp.float32),
            pltpu.VMEM((128, EROW), jnp.float32),
            pltpu.SemaphoreType.DMA,
            pltpu.SemaphoreType.DMA,
            pltpu.SemaphoreType.DMA,
            pltpu.SemaphoreType.DMA,
            pltpu.SemaphoreType.DMA,
        ],
    )
    return f(data_p, E_pad)


B_TILE = 512
S_CHUNK = 32


def _mlp_body(x_ref, w1_ref, b1_ref, w2_ref, b2_ref, w3_ref, b3_ref,
              mu_ref, sc_ref, acc_ref):
    k = pl.program_id(1)
    nk = pl.num_programs(1)

    @pl.when(k == 0)
    def _():
        acc_ref[...] = jnp.zeros_like(acc_ref)

    xb = x_ref[...].reshape(B_TILE, S_CHUNK * EROW)
    wb = w1_ref[...].reshape(S_CHUNK * EROW, 128)
    acc_ref[...] += jnp.dot(xb, wb, preferred_element_type=jnp.float32)

    @pl.when(k == nk - 1)
    def _():
        h1 = jnp.maximum(acc_ref[...] + b1_ref[...], 0.0)
        h2 = jnp.maximum(
            jnp.dot(h1, w2_ref[...], preferred_element_type=jnp.float32)
            + b2_ref[...], 0.0)
        out = jnp.dot(h2, w3_ref[...],
                      preferred_element_type=jnp.float32) + b3_ref[...]
        mu_ref[...] = out[:, :2]
        sc_ref[...] = jnp.logaddexp(out[:, 2:] - 5.0, 0.0)


def _mlp(x, W1p, b1, W2, b2, W3, b3):
    nb = HALF // B_TILE
    nk = SEQ_PAD // S_CHUNK
    mu, scale = pl.pallas_call(
        _mlp_body,
        grid=(nb, nk),
        in_specs=[
            pl.BlockSpec((B_TILE, S_CHUNK, EROW), lambda b, k: (b, k, 0)),
            pl.BlockSpec((S_CHUNK, EROW, 128), lambda b, k: (k, 0, 0)),
            pl.BlockSpec((128,), lambda b, k: (0,)),
            pl.BlockSpec((128, 64), lambda b, k: (0, 0)),
            pl.BlockSpec((64,), lambda b, k: (0,)),
            pl.BlockSpec((64, 4), lambda b, k: (0, 0)),
            pl.BlockSpec((4,), lambda b, k: (0,)),
        ],
        out_specs=[
            pl.BlockSpec((B_TILE, 2), lambda b, k: (b, 0)),
            pl.BlockSpec((B_TILE, 2), lambda b, k: (b, 0)),
        ],
        out_shape=[
            jax.ShapeDtypeStruct((HALF, 2), jnp.float32),
            jax.ShapeDtypeStruct((HALF, 2), jnp.float32),
        ],
        scratch_shapes=[pltpu.VMEM((B_TILE, 128), jnp.float32)],
    )(x, W1p, b1, W2, b2, W3, b3)
    return mu, scale


def kernel(data, E, W1, b1, W2, b2, W3, b3):
    data_p = jnp.pad(data, ((0, 0), (0, SEQ_PAD - SEQ)))
    data_p = data_p.reshape(BATCH, N_CHUNK, 128)
    E_pad = jnp.pad(E, ((0, 0), (0, EROW - EMB)))
    W1pp = jnp.pad(W1.reshape(SEQ, EMB, 128),
                   ((0, SEQ_PAD - SEQ), (0, EROW - EMB), (0, 0)))
    xs = [_sc_gather(data_p[i * HALF:(i + 1) * HALF], E_pad)
          for i in range(4)]
    outs = [_mlp(x, W1pp, b1, W2, b2, W3, b3) for x in xs]
    return (jnp.concatenate([o[0] for o in outs]),
            jnp.concatenate([o[1] for o in outs]))
